# Initial kernel scaffold; baseline (speedup 1.0000x reference)
import jax
import jax.numpy as jnp
from jax.experimental import pallas as pl


def kernel(input_ids, table, W, b):
    raise NotImplementedError("write your pallas kernel here")


# speedup vs baseline: 1.1082x; 1.1082x over previous
import functools

import jax
import jax.numpy as jnp
from jax import lax
from jax.experimental import pallas as pl
from jax.experimental.pallas import tpu as pltpu
from jax.experimental.pallas import tpu_sc as plsc


def _proj_body(t_ref, w_ref, b_ref, o_ref):
    o_ref[...] = jnp.maximum(
        jnp.dot(t_ref[...], w_ref[...], preferred_element_type=jnp.float32)
        + b_ref[...],
        0.0,
    )


def _project(table, W, b):
    V, E = table.shape
    O = W.shape[1]
    BR = 1000
    return pl.pallas_call(
        _proj_body,
        grid=(V // BR,),
        in_specs=[
            pl.BlockSpec((BR, E), lambda i: (i, 0)),
            pl.BlockSpec((E, O), lambda i: (0, 0)),
            pl.BlockSpec((1, O), lambda i: (0, 0)),
        ],
        out_specs=pl.BlockSpec((BR, O), lambda i: (i, 0)),
        out_shape=jax.ShapeDtypeStruct((V, O), jnp.float32),
    )(table, W, b.reshape(1, O))


@functools.lru_cache(maxsize=None)
def _make_gather(V, O, B, CW):
    info = plsc.get_sparse_core_info()
    NC, NS = info.num_cores, info.num_subcores
    NW = NC * NS
    assert B % (NW * CW) == 0
    rows_per_w = B // NW
    chunks = rows_per_w // CW
    mesh = plsc.VectorSubcoreMesh(core_axis_name="c", subcore_axis_name="s")

    @functools.partial(
        pl.kernel,
        mesh=mesh,
        out_type=jax.ShapeDtypeStruct((B, O), jnp.float32),
        scratch_types=[
            pltpu.VMEM((chunks, CW), jnp.int32),
            pltpu.VMEM((CW, O), jnp.float32),
            pltpu.SemaphoreType.DMA,
        ],
    )
    def gather(tbl_hbm, idx_hbm, out_hbm, idx_v, rows_v, sem):
        wid = lax.axis_index("s") * NC + lax.axis_index("c")
        pltpu.sync_copy(idx_hbm.at[pl.ds(wid * chunks, chunks)], idx_v)
        row0 = wid * rows_per_w

        def step(c, carry):
            pltpu.async_copy(tbl_hbm.at[idx_v.at[c]], rows_v, sem).wait()
            pltpu.sync_copy(rows_v, out_hbm.at[pl.ds(row0 + c * CW, CW)])
            return carry

        lax.fori_loop(0, chunks, step, 0)

    return gather


def kernel(input_ids, table, W, b):
    Bm, S = input_ids.shape
    V, E = table.shape
    O = W.shape[1]
    B = Bm * S
    CW = 64
    proj = _project(table, W, b)
    ids2 = input_ids.reshape(-1, CW).astype(jnp.int32)
    out = _make_gather(V, O, B, CW)(proj, ids2)
    return out.reshape(Bm, S, O)


# speedup vs baseline: 1.1837x; 1.0681x over previous
import functools

import jax
import jax.numpy as jnp
from jax import lax
from jax.experimental import pallas as pl
from jax.experimental.pallas import tpu as pltpu
from jax.experimental.pallas import tpu_sc as plsc


def _proj_body(t_ref, w_ref, b_ref, o_ref):
    o_ref[...] = jnp.maximum(
        jnp.dot(t_ref[...], w_ref[...], preferred_element_type=jnp.float32)
        + b_ref[...],
        0.0,
    )


def _project(table, W, b):
    V, E = table.shape
    O = W.shape[1]
    BR = 1000
    return pl.pallas_call(
        _proj_body,
        grid=(V // BR,),
        in_specs=[
            pl.BlockSpec((BR, E), lambda i: (i, 0)),
            pl.BlockSpec((E, O), lambda i: (0, 0)),
            pl.BlockSpec((1, O), lambda i: (0, 0)),
        ],
        out_specs=pl.BlockSpec((BR, O), lambda i: (i, 0)),
        out_shape=jax.ShapeDtypeStruct((V, O), jnp.float32),
    )(table, W, b.reshape(1, O))


@functools.lru_cache(maxsize=None)
def _make_gather(V, O, B, CW):
    info = plsc.get_sparse_core_info()
    NC, NS = info.num_cores, info.num_subcores
    NW = NC * NS
    assert B % (NW * CW) == 0
    rows_per_w = B // NW
    chunks = rows_per_w // CW
    mesh = plsc.VectorSubcoreMesh(core_axis_name="c", subcore_axis_name="s")

    @functools.partial(
        pl.kernel,
        mesh=mesh,
        out_type=jax.ShapeDtypeStruct((B, O), jnp.float32),
        scratch_types=[
            pltpu.VMEM((chunks, CW), jnp.int32),
            pltpu.VMEM((2, CW, O), jnp.float32),
            pltpu.SemaphoreType.DMA,
            pltpu.SemaphoreType.DMA,
        ],
    )
    def gather(tbl_hbm, idx_hbm, out_hbm, idx_v, rows_v, sem0, sem1):
        wid = lax.axis_index("s") * NC + lax.axis_index("c")
        sems = (sem0, sem1)
        pltpu.sync_copy(idx_hbm.at[pl.ds(wid * chunks, chunks)], idx_v)
        row0 = wid * rows_per_w

        for b in range(2):
            pltpu.async_copy(tbl_hbm.at[idx_v.at[b]], rows_v.at[b], sems[b])

        def step(i, carry):
            c = i * 2
            for b in range(2):
                ch = c + b
                pltpu.make_async_copy(
                    tbl_hbm.at[idx_v.at[ch]], rows_v.at[b], sems[b]
                ).wait()
                pltpu.sync_copy(
                    rows_v.at[b], out_hbm.at[pl.ds(row0 + ch * CW, CW)]
                )

                @pl.when(ch + 2 < chunks)
                def _():
                    pltpu.async_copy(
                        tbl_hbm.at[idx_v.at[ch + 2]], rows_v.at[b], sems[b]
                    )

            return carry

        lax.fori_loop(0, chunks // 2, step, 0)

    return gather


def kernel(input_ids, table, W, b):
    Bm, S = input_ids.shape
    V, E = table.shape
    O = W.shape[1]
    B = Bm * S
    CW = 64
    proj = _project(table, W, b)
    ids2 = input_ids.reshape(-1, CW).astype(jnp.int32)
    out = _make_gather(V, O, B, CW)(proj, ids2)
    return out.reshape(Bm, S, O)
